# baseline (speedup 1.0000x reference)
import functools

import jax
import jax.numpy as jnp
from jax.experimental import pallas as pl
from jax.experimental.pallas import tpu as pltpu

_NB = 10
_A = 0.1
_GROUP = 7 * 8
_ROUND = 32 * 8
_MASK_E = 0o0707070707
_CH = 1024


def _loss_kernel(in_ref, tgt_ref, out_ref, mse_ref, lo_ref, hi_ref,
                 macc_ref, mnacc_ref, mxacc_ref, hist_ref,
                 *, nblk, br, cols, total):
    p = pl.program_id(0)
    i = pl.program_id(1)

    @pl.when((p == 0) & (i == 0))
    def _init():
        macc_ref[...] = jnp.zeros(macc_ref.shape, macc_ref.dtype)
        mnacc_ref[...] = jnp.full(mnacc_ref.shape, jnp.inf, mnacc_ref.dtype)
        mxacc_ref[...] = jnp.full(mxacc_ref.shape, -jnp.inf, mxacc_ref.dtype)
        hist_ref[...] = jnp.zeros(hist_ref.shape, hist_ref.dtype)

    @pl.when(p == 0)
    def _pass0():
        for c0 in range(0, cols, _CH):
            m = macc_ref[:, c0:c0 + _CH]
            mn = mnacc_ref[:, c0:c0 + _CH]
            mx = mxacc_ref[:, c0:c0 + _CH]
            for r in range(0, br, 8):
                x = in_ref[r:r + 8, c0:c0 + _CH]
                d = x - tgt_ref[r:r + 8, c0:c0 + _CH]
                m = m + d * d
                mn = jnp.minimum(mn, x)
                mx = jnp.maximum(mx, x)
            macc_ref[:, c0:c0 + _CH] = m
            mnacc_ref[:, c0:c0 + _CH] = mn
            mxacc_ref[:, c0:c0 + _CH] = mx

    @pl.when((p == 0) & (i == nblk - 1))
    def _minmax():
        mse_ref[0] = jnp.sum(macc_ref[...])
        lo_ref[0] = jnp.min(mnacc_ref[...])
        hi_ref[0] = jnp.max(mxacc_ref[...])

    @pl.when(p == 1)
    def _pass1():
        lo = lo_ref[0]
        a = _NB / (hi_ref[0] - lo)
        b = -(lo * a)
        top = jnp.float32(_NB - 1)
        lanes_c = _CH // 128
        for c0 in range(0, cols, _CH):
            for q0 in range(0, br, _ROUND):
                q1 = min(q0 + _ROUND, br)
                e2 = jnp.zeros((8, _CH), jnp.int32)
                o2 = jnp.zeros((8, _CH), jnp.int32)
                r0 = q0
                while r0 < q1:
                    r1 = min(r0 + _GROUP, q1)
                    acc = jnp.zeros((8, _CH), jnp.int32)
                    for r in range(r0, r1, 8):
                        x = in_ref[r:r + 8, c0:c0 + _CH]
                        idx = jnp.minimum(x * a + b, top).astype(jnp.int32)
                        acc = acc + jnp.left_shift(jnp.int32(1),
                                                   idx + idx + idx)
                    e2 = e2 + (acc & _MASK_E)
                    o2 = o2 + (jnp.right_shift(acc, 3) & _MASK_E)
                    r0 = r1
                for k in range(5):
                    fe = jnp.right_shift(e2, 6 * k) & 63
                    fo = jnp.right_shift(o2, 6 * k) & 63
                    hist_ref[8 * (2 * k):8 * (2 * k) + 8, :] += jnp.sum(
                        fe.reshape(8, lanes_c, 128), axis=1)
                    hist_ref[8 * (2 * k + 1):8 * (2 * k + 1) + 8, :] += jnp.sum(
                        fo.reshape(8, lanes_c, 128), axis=1)

    @pl.when((p == 1) & (i == nblk - 1))
    def _finish():
        counts = jnp.sum(
            hist_ref[...].reshape(_NB, 8 * 128).astype(jnp.float32), axis=1)
        h = counts / float(total)
        entropy = -jnp.sum(h * jnp.log(h + 1e-09))
        out_ref[0] = mse_ref[0] / float(total) - _A * entropy


def kernel(input, target):
    rows, cols = input.shape
    br = 512 if rows % 512 == 0 else rows
    nblk = rows // br
    total = rows * cols

    out = pl.pallas_call(
        functools.partial(_loss_kernel, nblk=nblk, br=br, cols=cols,
                          total=total),
        grid=(2, nblk),
        in_specs=[
            pl.BlockSpec((br, cols), lambda p, i: (i, 0)),
            pl.BlockSpec((br, cols), lambda p, i: (i * (1 - p), 0)),
        ],
        out_specs=pl.BlockSpec(memory_space=pltpu.SMEM),
        out_shape=jax.ShapeDtypeStruct((1,), jnp.float32),
        scratch_shapes=[
            pltpu.SMEM((1,), jnp.float32),
            pltpu.SMEM((1,), jnp.float32),
            pltpu.SMEM((1,), jnp.float32),
            pltpu.VMEM((8, cols), jnp.float32),
            pltpu.VMEM((8, cols), jnp.float32),
            pltpu.VMEM((8, cols), jnp.float32),
            pltpu.VMEM((_NB * 8, 128), jnp.int32),
        ],
    )(input, target)
    return out[0]


# speedup vs baseline: 1.0356x; 1.0356x over previous
import functools

import jax
import jax.numpy as jnp
from jax.experimental import pallas as pl
from jax.experimental.pallas import tpu as pltpu

_NB = 10
_A = 0.1
_GROUP = 7 * 8
_ROUND = 32 * 8
_MASK_E = 0o0707070707
_CH = 1024


def _loss_kernel(in_ref, tgt_ref, out_ref, mse_ref, lo_ref, hi_ref,
                 macc_ref, mnacc_ref, mxacc_ref, hist_ref,
                 *, nblk, br, cols, total):
    p = pl.program_id(0)
    i = pl.program_id(1)

    @pl.when((p == 0) & (i == 0))
    def _init():
        macc_ref[...] = jnp.zeros(macc_ref.shape, macc_ref.dtype)
        mnacc_ref[...] = jnp.full(mnacc_ref.shape, jnp.inf, mnacc_ref.dtype)
        mxacc_ref[...] = jnp.full(mxacc_ref.shape, -jnp.inf, mxacc_ref.dtype)
        hist_ref[...] = jnp.zeros(hist_ref.shape, hist_ref.dtype)

    @pl.when(p == 0)
    def _pass0():
        for c0 in range(0, cols, _CH):
            m = macc_ref[:, c0:c0 + _CH]
            mn = mnacc_ref[:, c0:c0 + _CH]
            mx = mxacc_ref[:, c0:c0 + _CH]
            for r in range(0, br, 8):
                x = in_ref[r:r + 8, c0:c0 + _CH]
                d = x - tgt_ref[r:r + 8, c0:c0 + _CH]
                m = m + d * d
                mn = jnp.minimum(mn, x)
                mx = jnp.maximum(mx, x)
            macc_ref[:, c0:c0 + _CH] = m
            mnacc_ref[:, c0:c0 + _CH] = mn
            mxacc_ref[:, c0:c0 + _CH] = mx

    @pl.when((p == 0) & (i == nblk - 1))
    def _minmax():
        mse_ref[0] = jnp.sum(macc_ref[...])
        lo_ref[0] = jnp.min(mnacc_ref[...])
        hi_ref[0] = jnp.max(mxacc_ref[...])

    @pl.when(p == 1)
    def _pass1():
        lo = lo_ref[0]
        a = _NB / (hi_ref[0] - lo)
        b = -(lo * a)
        top = jnp.float32(_NB - 1)
        lanes_c = _CH // 128
        for c0 in range(0, cols, _CH):
            for q0 in range(0, br, _ROUND):
                q1 = min(q0 + _ROUND, br)
                e2 = jnp.zeros((8, _CH), jnp.int32)
                o2 = jnp.zeros((8, _CH), jnp.int32)
                r0 = q0
                while r0 < q1:
                    r1 = min(r0 + _GROUP, q1)
                    acc = jnp.zeros((8, _CH), jnp.int32)
                    for r in range(r0, r1, 8):
                        x = in_ref[r:r + 8, c0:c0 + _CH]
                        idx = jnp.minimum(x * a + b, top).astype(jnp.int32)
                        acc = acc + jnp.left_shift(jnp.int32(1),
                                                   idx + idx + idx)
                    e2 = e2 + (acc & _MASK_E)
                    o2 = o2 + (jnp.right_shift(acc, 3) & _MASK_E)
                    r0 = r1
                for k in range(5):
                    fe = jnp.right_shift(e2, 6 * k) & 63
                    fo = jnp.right_shift(o2, 6 * k) & 63
                    hist_ref[8 * (2 * k):8 * (2 * k) + 8, :] += jnp.sum(
                        fe.reshape(8, lanes_c, 128), axis=1)
                    hist_ref[8 * (2 * k + 1):8 * (2 * k + 1) + 8, :] += jnp.sum(
                        fo.reshape(8, lanes_c, 128), axis=1)

    @pl.when((p == 1) & (i == nblk - 1))
    def _finish():
        counts = jnp.sum(
            hist_ref[...].reshape(_NB, 8 * 128).astype(jnp.float32), axis=1)
        h = counts / float(total)
        entropy = -jnp.sum(h * jnp.log(h + 1e-09))
        out_ref[0] = mse_ref[0] / float(total) - _A * entropy


def kernel(input, target):
    rows, cols = input.shape
    br = 256 if rows % 256 == 0 else rows
    nblk = rows // br
    total = rows * cols

    out = pl.pallas_call(
        functools.partial(_loss_kernel, nblk=nblk, br=br, cols=cols,
                          total=total),
        grid=(2, nblk),
        in_specs=[
            pl.BlockSpec((br, cols), lambda p, i: (i, 0)),
            pl.BlockSpec((br, cols), lambda p, i: (i * (1 - p), 0)),
        ],
        out_specs=pl.BlockSpec(memory_space=pltpu.SMEM),
        out_shape=jax.ShapeDtypeStruct((1,), jnp.float32),
        scratch_shapes=[
            pltpu.SMEM((1,), jnp.float32),
            pltpu.SMEM((1,), jnp.float32),
            pltpu.SMEM((1,), jnp.float32),
            pltpu.VMEM((8, cols), jnp.float32),
            pltpu.VMEM((8, cols), jnp.float32),
            pltpu.VMEM((8, cols), jnp.float32),
            pltpu.VMEM((_NB * 8, 128), jnp.int32),
        ],
    )(input, target)
    return out[0]


# speedup vs baseline: 1.0455x; 1.0096x over previous
import functools

import jax
import jax.numpy as jnp
from jax.experimental import pallas as pl
from jax.experimental.pallas import tpu as pltpu

_NB = 10
_A = 0.1
_GROUP = 7 * 8
_ROUND = 32 * 8
_MASK_E = 0o0707070707
_CH = 1024


def _loss_kernel(in_ref, tgt_ref, out_ref, mse_ref, lo_ref, hi_ref,
                 macc_ref, mnacc_ref, mxacc_ref, hist_ref,
                 *, nblk, br, cols, total):
    p = pl.program_id(0)
    i = pl.program_id(1)

    @pl.when((p == 0) & (i == 0))
    def _init():
        macc_ref[...] = jnp.zeros(macc_ref.shape, macc_ref.dtype)
        mnacc_ref[...] = jnp.full(mnacc_ref.shape, jnp.inf, mnacc_ref.dtype)
        mxacc_ref[...] = jnp.full(mxacc_ref.shape, -jnp.inf, mxacc_ref.dtype)
        hist_ref[...] = jnp.zeros(hist_ref.shape, hist_ref.dtype)

    @pl.when(p == 0)
    def _pass0():
        for c0 in range(0, cols, _CH):
            m = macc_ref[:, c0:c0 + _CH]
            mn = mnacc_ref[:, c0:c0 + _CH]
            mx = mxacc_ref[:, c0:c0 + _CH]
            for r in range(0, br, 8):
                x = in_ref[r:r + 8, c0:c0 + _CH]
                d = x - tgt_ref[r:r + 8, c0:c0 + _CH]
                m = m + d * d
                mn = jnp.minimum(mn, x)
                mx = jnp.maximum(mx, x)
            macc_ref[:, c0:c0 + _CH] = m
            mnacc_ref[:, c0:c0 + _CH] = mn
            mxacc_ref[:, c0:c0 + _CH] = mx

    @pl.when((p == 0) & (i == nblk - 1))
    def _minmax():
        mse_ref[0] = jnp.sum(macc_ref[...])
        lo_ref[0] = jnp.min(mnacc_ref[...])
        hi_ref[0] = jnp.max(mxacc_ref[...])

    @pl.when(p == 1)
    def _pass1():
        lo = lo_ref[0]
        a = _NB / (hi_ref[0] - lo)
        b = -(lo * a)
        top = jnp.float32(_NB - 1)
        _CH1 = 2048
        lanes_c = _CH1 // 128
        for c0 in range(0, cols, _CH1):
            for q0 in range(0, br, _ROUND):
                q1 = min(q0 + _ROUND, br)
                e2 = jnp.zeros((8, _CH1), jnp.int32)
                o2 = jnp.zeros((8, _CH1), jnp.int32)
                r0 = q0
                while r0 < q1:
                    r1 = min(r0 + _GROUP, q1)
                    acc = jnp.zeros((8, _CH1), jnp.int32)
                    for r in range(r0, r1, 8):
                        x = in_ref[r:r + 8, c0:c0 + _CH1]
                        idx = jnp.minimum(x * a + b, top).astype(jnp.int32)
                        acc = acc + jnp.left_shift(jnp.int32(1),
                                                   idx + idx + idx)
                    e2 = e2 + (acc & _MASK_E)
                    o2 = o2 + (jnp.right_shift(acc, 3) & _MASK_E)
                    r0 = r1
                for k in range(5):
                    fe = jnp.right_shift(e2, 6 * k) & 63
                    fo = jnp.right_shift(o2, 6 * k) & 63
                    hist_ref[8 * (2 * k):8 * (2 * k) + 8, :] += jnp.sum(
                        fe.reshape(8, lanes_c, 128), axis=1)
                    hist_ref[8 * (2 * k + 1):8 * (2 * k + 1) + 8, :] += jnp.sum(
                        fo.reshape(8, lanes_c, 128), axis=1)

    @pl.when((p == 1) & (i == nblk - 1))
    def _finish():
        counts = jnp.sum(
            hist_ref[...].reshape(_NB, 8 * 128).astype(jnp.float32), axis=1)
        h = counts / float(total)
        entropy = -jnp.sum(h * jnp.log(h + 1e-09))
        out_ref[0] = mse_ref[0] / float(total) - _A * entropy


def kernel(input, target):
    rows, cols = input.shape
    br = 256 if rows % 256 == 0 else rows
    nblk = rows // br
    total = rows * cols

    out = pl.pallas_call(
        functools.partial(_loss_kernel, nblk=nblk, br=br, cols=cols,
                          total=total),
        grid=(2, nblk),
        in_specs=[
            pl.BlockSpec((br, cols), lambda p, i: (i, 0)),
            pl.BlockSpec((br, cols), lambda p, i: (i * (1 - p), 0)),
        ],
        out_specs=pl.BlockSpec(memory_space=pltpu.SMEM),
        out_shape=jax.ShapeDtypeStruct((1,), jnp.float32),
        scratch_shapes=[
            pltpu.SMEM((1,), jnp.float32),
            pltpu.SMEM((1,), jnp.float32),
            pltpu.SMEM((1,), jnp.float32),
            pltpu.VMEM((8, cols), jnp.float32),
            pltpu.VMEM((8, cols), jnp.float32),
            pltpu.VMEM((8, cols), jnp.float32),
            pltpu.VMEM((_NB * 8, 128), jnp.int32),
        ],
    )(input, target)
    return out[0]
